# baseline (speedup 1.0000x reference)
import functools

import jax
import jax.numpy as jnp
from jax import lax
from jax.experimental import pallas as pl
from jax.experimental.pallas import tpu as pltpu
from jax.experimental.pallas import tpu_sc as plsc

E = 8
D = 768
H = 192
N = 4096
M = 256
NP = N + E * M
NBLK = NP // M
CCH = 512

NC = 2
NS = 16
NW = NC * NS
RPW = N // NW


def _router_kernel(x_ref, rw_ref, rb_ref, pos_ref, pb_ref):
    logits = lax.dot_general(rw_ref[...], x_ref[...],
                             (((1,), (1,)), ((), ())),
                             preferred_element_type=jnp.float32)
    logits = logits + rb_ref[...].reshape(E, 1)
    m = jnp.max(logits, axis=0, keepdims=True)
    iota_e = lax.broadcasted_iota(jnp.int32, (E, N), 0)
    eid = jnp.min(jnp.where(logits == m, iota_e, E), axis=0, keepdims=True)
    onehot = (iota_e == eid).astype(jnp.float32)
    ci = lax.broadcasted_iota(jnp.int32, (CCH, CCH), 0)
    cj = lax.broadcasted_iota(jnp.int32, (CCH, CCH), 1)
    tri = (ci <= cj).astype(jnp.float32)
    chunks = []
    carry = jnp.zeros((E, 1), dtype=jnp.float32)
    for c in range(N // CCH):
        oh_c = onehot[:, c * CCH:(c + 1) * CCH]
        cum_c = jnp.dot(oh_c, tri, preferred_element_type=jnp.float32,
                        precision=lax.Precision.HIGHEST) + carry
        chunks.append(cum_c)
        carry = cum_c[:, CCH - 1:CCH]
    cum = jnp.concatenate(chunks, axis=1)
    counts = (carry + 0.5).astype(jnp.int32)
    nblk_e = ((counts + (M - 1)) // M).astype(jnp.float32)
    li = lax.broadcasted_iota(jnp.int32, (E + 1, E), 0)
    lj = lax.broadcasted_iota(jnp.int32, (E + 1, E), 1)
    lower = (lj < li).astype(jnp.float32)
    pb = jnp.dot(lower, nblk_e, preferred_element_type=jnp.float32,
                 precision=lax.Precision.HIGHEST)
    base = pb[0:E] * float(M)
    pos = jnp.sum(onehot * (cum + base), axis=0, keepdims=True) - 1.0
    pos_ref[...] = (pos + 0.5).astype(jnp.int32)
    pb_ref[...] = (pb + 0.5).astype(jnp.int32)


def _grouped_ffn_kernel(pb_ref, xs_ref, w1_ref, b1_ref, w2_ref, b2_ref,
                        out_ref):
    i = pl.program_id(0)

    @pl.when(i < pb_ref[E])
    def _():
        h = jnp.dot(xs_ref[...], w1_ref[0],
                    preferred_element_type=jnp.float32)
        h = jnp.maximum(h + b1_ref[0], 0.0)
        y = jnp.dot(h, w2_ref[0], preferred_element_type=jnp.float32)
        out_ref[...] = jnp.maximum(y + b2_ref[0], 0.0)


def _expert_of_block(i, pb_ref):
    e = 0
    for k in range(1, E):
        e += jnp.where(pb_ref[k] <= i, 1, 0)
    return e


def _sc_scatter_kernel(x_hbm, pos_hbm, xs_hbm, idx_v, rows_v, sem):
    wid = lax.axis_index("s") * NC + lax.axis_index("c")
    base = wid * RPW
    pltpu.sync_copy(pos_hbm.at[wid], idx_v)
    pltpu.sync_copy(x_hbm.at[pl.ds(base, RPW)], rows_v)
    pltpu.async_copy(rows_v, xs_hbm.at[idx_v.at[0]], sem).wait()


def _sc_gather_kernel(ys_hbm, pos_hbm, out_hbm, idx_v, rows_v, sem):
    wid = lax.axis_index("s") * NC + lax.axis_index("c")
    base = wid * RPW
    pltpu.sync_copy(pos_hbm.at[wid], idx_v)
    pltpu.async_copy(ys_hbm.at[idx_v.at[0]], rows_v, sem).wait()
    pltpu.sync_copy(rows_v, out_hbm.at[pl.ds(base, RPW)])


def _sc_mesh():
    return plsc.VectorSubcoreMesh(core_axis_name="c", subcore_axis_name="s")


def kernel(x, router_w, router_b, w1, b1, w2, b2):
    B, K, Dq = x.shape
    x_flat = x.reshape(N, D)

    pos, pb = pl.pallas_call(
        _router_kernel,
        in_specs=[
            pl.BlockSpec((N, D), lambda: (0, 0)),
            pl.BlockSpec((E, D), lambda: (0, 0)),
            pl.BlockSpec((1, E), lambda: (0, 0)),
        ],
        out_specs=[
            pl.BlockSpec((1, N), lambda: (0, 0)),
            pl.BlockSpec((E + 1, 1), lambda: (0, 0)),
        ],
        out_shape=[
            jax.ShapeDtypeStruct((1, N), jnp.int32),
            jax.ShapeDtypeStruct((E + 1, 1), jnp.int32),
        ],
    )(x_flat, router_w, router_b.reshape(1, E))

    pos3 = pos.reshape(NW, 1, RPW)
    pb9 = pb.reshape(E + 1)

    scatter = pl.kernel(
        _sc_scatter_kernel,
        out_type=jax.ShapeDtypeStruct((NP, D), jnp.float32),
        mesh=_sc_mesh(),
        scratch_types=[
            pltpu.VMEM((1, RPW), jnp.int32),
            pltpu.VMEM((RPW, D), jnp.float32),
            pltpu.SemaphoreType.DMA,
        ],
    )
    xs = scatter(x_flat, pos3)

    ys = pl.pallas_call(
        _grouped_ffn_kernel,
        grid_spec=pltpu.PrefetchScalarGridSpec(
            num_scalar_prefetch=1,
            grid=(NBLK,),
            in_specs=[
                pl.BlockSpec((M, D), lambda i, pb: (i, 0)),
                pl.BlockSpec((1, D, H),
                             lambda i, pb: (_expert_of_block(i, pb), 0, 0)),
                pl.BlockSpec((1, 1, H),
                             lambda i, pb: (_expert_of_block(i, pb), 0, 0)),
                pl.BlockSpec((1, H, D),
                             lambda i, pb: (_expert_of_block(i, pb), 0, 0)),
                pl.BlockSpec((1, 1, D),
                             lambda i, pb: (_expert_of_block(i, pb), 0, 0)),
            ],
            out_specs=pl.BlockSpec((M, D), lambda i, pb: (i, 0)),
        ),
        out_shape=jax.ShapeDtypeStruct((NP, D), jnp.float32),
    )(pb9, xs, w1, b1.reshape(E, 1, H), w2, b2.reshape(E, 1, D))

    gather = pl.kernel(
        _sc_gather_kernel,
        out_type=jax.ShapeDtypeStruct((N, D), jnp.float32),
        mesh=_sc_mesh(),
        scratch_types=[
            pltpu.VMEM((1, RPW), jnp.int32),
            pltpu.VMEM((RPW, D), jnp.float32),
            pltpu.SemaphoreType.DMA,
        ],
    )
    out = gather(ys, pos3)
    return out.reshape(B, K, Dq)
